# baseline (speedup 1.0000x reference)
import dataclasses
import functools

import jax
import jax.numpy as jnp
from jax import lax
from jax.experimental import pallas as pl
from jax.experimental.pallas import tpu as pltpu
from jax.experimental.pallas import tpu_sc as plsc

DIM = 64
BATCH = 16384
NEG = 20

NCORE = 2
NSUB = 16
LANES = 16
NWORK = NCORE * NSUB
ROWS_W = BATCH // NWORK
CH = 32
NCHUNK = ROWS_W // CH
NEG_CH = CH * NEG
NEG_SLAB = 128
NSLAB = NEG_CH // NEG_SLAB


def _log_sigmoid(x):
    e = jnp.exp(-jnp.abs(x))
    t = e / (2.0 + e)
    t2 = t * t
    ln = 2.0 * t * (1.0 + t2 * (1.0 / 3.0 + t2 * (0.2 + t2 * (1.0 / 7.0 + t2 * (1.0 / 9.0)))))
    return jnp.minimum(x, 0.0) - ln


def _sc_partial_loss(v_i, v_j, neg2d, nodes, ctx):
    mesh = plsc.VectorSubcoreMesh(core_axis_name="c", subcore_axis_name="s")
    cp = pltpu.CompilerParams(needs_layout_passes=False,
                              use_tc_tiling_on_sc=False)

    @functools.partial(
        pl.kernel,
        mesh=mesh,
        compiler_params=cp,
        out_type=jax.ShapeDtypeStruct((NWORK * LANES,), jnp.float32),
        scratch_types=[
            pltpu.VMEM((CH,), jnp.int32),
            pltpu.VMEM((CH,), jnp.int32),
            pltpu.VMEM((NEG_CH,), jnp.int32),
            pltpu.VMEM((CH, DIM), jnp.float32),
            pltpu.VMEM((CH, DIM), jnp.float32),
            pltpu.VMEM((NEG_CH, DIM), jnp.float32),
            pltpu.VMEM((LANES,), jnp.float32),
            pltpu.SemaphoreType.DMA,
        ],
    )
    def sc_kernel(vi_hbm, vj_hbm, neg_hbm, nodes_hbm, ctx_hbm, out_hbm,
                  vi_idx, vj_idx, neg_idx, vi_rows, vj_rows, neg_rows,
                  loss_v, sem):
        wid = lax.axis_index("s") * NCORE + lax.axis_index("c")
        lane = lax.iota(jnp.int32, LANES)

        def chunk_body(c, loss):
            base = wid * ROWS_W + c * CH
            pltpu.sync_copy(vi_hbm.at[pl.ds(base, CH)], vi_idx)
            pltpu.sync_copy(vj_hbm.at[pl.ds(base, CH)], vj_idx)
            nbase = wid * ROWS_W * NEG + c * NEG_CH
            pltpu.sync_copy(neg_hbm.at[pl.ds(nbase, NEG_CH)], neg_idx)
            copies = [
                pltpu.async_copy(nodes_hbm.at[vi_idx], vi_rows, sem),
                pltpu.async_copy(ctx_hbm.at[vj_idx], vj_rows, sem),
            ]
            for j in range(NSLAB):
                copies.append(pltpu.async_copy(
                    ctx_hbm.at[neg_idx.at[pl.ds(j * NEG_SLAB, NEG_SLAB)]],
                    neg_rows.at[pl.ds(j * NEG_SLAB, NEG_SLAB)], sem))
            for cp in copies:
                cp.wait()

            for g in range(CH // LANES):
                row = lane + g * LANES
                nrows = [row * NEG + k for k in range(NEG)]

                def dot_body(d, accs):
                    col = jnp.full((LANES,), 0, jnp.int32) + d
                    vi_d = plsc.load_gather(vi_rows, [row, col])
                    vj_d = plsc.load_gather(vj_rows, [row, col])
                    out = [accs[0] + vi_d * vj_d]
                    for k in range(NEG):
                        nv = plsc.load_gather(neg_rows, [nrows[k], col])
                        out.append(accs[k + 1] + vi_d * nv)
                    return tuple(out)

                zero = jnp.zeros((LANES,), jnp.float32)
                accs = lax.fori_loop(0, DIM, dot_body, (zero,) * (NEG + 1))
                ls = _log_sigmoid(accs[0])
                for k in range(NEG):
                    ls = ls + _log_sigmoid(-accs[k + 1])
                loss = loss + ls
            return loss

        loss = lax.fori_loop(0, NCHUNK, chunk_body,
                             jnp.zeros((LANES,), jnp.float32))
        loss_v[...] = loss
        pltpu.sync_copy(loss_v, out_hbm.at[pl.ds(wid * LANES, LANES)])

    return sc_kernel(v_i, v_j, neg2d, nodes, ctx)


def _tc_finish(partials):
    def body(p_ref, o_ref):
        o_ref[0, 0] = -jnp.sum(p_ref[...]) * (1.0 / BATCH)

    return pl.pallas_call(
        body,
        out_shape=jax.ShapeDtypeStruct((1, 1), jnp.float32),
        out_specs=pl.BlockSpec(memory_space=pltpu.SMEM),
    )(partials)


def kernel(v_i, v_j, negsamples, nodes_embeddings, context_nodes_embed):
    negflat = negsamples.reshape(BATCH * NEG)
    partials = _sc_partial_loss(v_i.astype(jnp.int32), v_j.astype(jnp.int32),
                                negflat.astype(jnp.int32),
                                nodes_embeddings, context_nodes_embed)
    return _tc_finish(partials.reshape(NWORK, LANES))[0, 0]


# speedup vs baseline: 1.0025x; 1.0025x over previous
import functools

import jax
import jax.numpy as jnp
from jax import lax
from jax.experimental import pallas as pl
from jax.experimental.pallas import tpu as pltpu
from jax.experimental.pallas import tpu_sc as plsc

SIZE = 1000000
DIM = 64
BATCH = 16384
NEG = 20

NCORE = 2
NSUB = 16
LANES = 16
NWORK = NCORE * NSUB
ROWS_W = BATCH // NWORK
CH = 32
NCHUNK = ROWS_W // CH
NEG_CH = CH * NEG
NEG_SLAB = 128
NSLAB = NEG_CH // NEG_SLAB
PDIM = 2 * DIM


def _log_sigmoid(x):
    e = jnp.exp(-jnp.abs(x))
    t = e / (2.0 + e)
    t2 = t * t
    ln = 2.0 * t * (1.0 + t2 * (1.0 / 3.0 + t2 * (0.2 + t2 * (1.0 / 7.0 + t2 * (1.0 / 9.0)))))
    return jnp.minimum(x, 0.0) - ln


def _sc_partial_loss(v_i, v_j, negflat, nodes2, ctx2):
    mesh = plsc.VectorSubcoreMesh(core_axis_name="c", subcore_axis_name="s")
    cp = pltpu.CompilerParams(needs_layout_passes=False)

    @functools.partial(
        pl.kernel,
        mesh=mesh,
        compiler_params=cp,
        out_type=jax.ShapeDtypeStruct((NWORK * LANES,), jnp.float32),
        scratch_types=[
            pltpu.VMEM((CH,), jnp.int32),
            pltpu.VMEM((CH,), jnp.int32),
            pltpu.VMEM((NEG_CH,), jnp.int32),
            pltpu.VMEM((CH,), jnp.int32),
            pltpu.VMEM((CH,), jnp.int32),
            pltpu.VMEM((NEG_CH,), jnp.int32),
            pltpu.VMEM((CH, PDIM), jnp.float32),
            pltpu.VMEM((CH, PDIM), jnp.float32),
            pltpu.VMEM((NEG_CH, PDIM), jnp.float32),
            pltpu.VMEM((LANES,), jnp.float32),
            pltpu.SemaphoreType.DMA,
        ],
    )
    def sc_kernel(vi_hbm, vj_hbm, neg_hbm, nodes_hbm, ctx_hbm, out_hbm,
                  vi_idx, vj_idx, neg_idx, vi_pair, vj_pair, neg_pair,
                  vi_rows, vj_rows, neg_rows, loss_v, sem):
        wid = lax.axis_index("s") * NCORE + lax.axis_index("c")
        lane = lax.iota(jnp.int32, LANES)

        def chunk_body(c, loss):
            base = wid * ROWS_W + c * CH
            pltpu.sync_copy(vi_hbm.at[pl.ds(base, CH)], vi_idx)
            pltpu.sync_copy(vj_hbm.at[pl.ds(base, CH)], vj_idx)
            nbase = wid * ROWS_W * NEG + c * NEG_CH
            pltpu.sync_copy(neg_hbm.at[pl.ds(nbase, NEG_CH)], neg_idx)
            for t in range(CH // LANES):
                sl = pl.ds(t * LANES, LANES)
                vi_pair[sl] = vi_idx[sl] >> 1
                vj_pair[sl] = vj_idx[sl] >> 1
            for t in range(NEG_CH // LANES):
                sl = pl.ds(t * LANES, LANES)
                neg_pair[sl] = neg_idx[sl] >> 1
            copies = [
                pltpu.async_copy(nodes_hbm.at[vi_pair], vi_rows, sem),
                pltpu.async_copy(ctx_hbm.at[vj_pair], vj_rows, sem),
            ]
            for j in range(NSLAB):
                copies.append(pltpu.async_copy(
                    ctx_hbm.at[neg_pair.at[pl.ds(j * NEG_SLAB, NEG_SLAB)]],
                    neg_rows.at[pl.ds(j * NEG_SLAB, NEG_SLAB)], sem))
            for cpy in copies:
                cpy.wait()

            for g in range(CH // LANES):
                row = lane + g * LANES
                nrows = [row * NEG + k for k in range(NEG)]
                vi_off = (plsc.load_gather(vi_idx, [row]) & 1) * DIM
                vj_off = (plsc.load_gather(vj_idx, [row]) & 1) * DIM
                ng_off = [(plsc.load_gather(neg_idx, [nrows[k]]) & 1) * DIM
                          for k in range(NEG)]

                def dot_body(d, accs):
                    vi_d = plsc.load_gather(vi_rows, [row, vi_off + d])
                    vj_d = plsc.load_gather(vj_rows, [row, vj_off + d])
                    out = [accs[0] + vi_d * vj_d]
                    for k in range(NEG):
                        nv = plsc.load_gather(neg_rows, [nrows[k], ng_off[k] + d])
                        out.append(accs[k + 1] + vi_d * nv)
                    return tuple(out)

                zero = jnp.zeros((LANES,), jnp.float32)
                accs = lax.fori_loop(0, DIM, dot_body, (zero,) * (NEG + 1))
                ls = _log_sigmoid(accs[0])
                for k in range(NEG):
                    ls = ls + _log_sigmoid(-accs[k + 1])
                loss = loss + ls
            return loss

        loss = lax.fori_loop(0, NCHUNK, chunk_body,
                             jnp.zeros((LANES,), jnp.float32))
        loss_v[...] = loss
        pltpu.sync_copy(loss_v, out_hbm.at[pl.ds(wid * LANES, LANES)])

    return sc_kernel(v_i, v_j, negflat, nodes2, ctx2)


def _tc_finish(partials):
    def body(p_ref, o_ref):
        o_ref[0, 0] = -jnp.sum(p_ref[...]) * (1.0 / BATCH)

    return pl.pallas_call(
        body,
        out_shape=jax.ShapeDtypeStruct((1, 1), jnp.float32),
        out_specs=pl.BlockSpec(memory_space=pltpu.SMEM),
    )(partials)


def kernel(v_i, v_j, negsamples, nodes_embeddings, context_nodes_embed):
    negflat = negsamples.reshape(BATCH * NEG)
    nodes2 = nodes_embeddings.reshape(SIZE // 2, PDIM)
    ctx2 = context_nodes_embed.reshape(SIZE // 2, PDIM)
    partials = _sc_partial_loss(v_i.astype(jnp.int32), v_j.astype(jnp.int32),
                                negflat.astype(jnp.int32), nodes2, ctx2)
    return _tc_finish(partials.reshape(NWORK, LANES))[0, 0]


# speedup vs baseline: 1.0494x; 1.0467x over previous
import functools

import jax
import jax.numpy as jnp
from jax import lax
from jax.experimental import pallas as pl
from jax.experimental.pallas import tpu as pltpu
from jax.experimental.pallas import tpu_sc as plsc

SIZE = 1000000
DIM = 64
BATCH = 16384
NEG = 20

NCORE = 2
NSUB = 16
LANES = 16
NWORK = NCORE * NSUB
ROWS_W = BATCH // NWORK
NEG_W = ROWS_W * NEG
CH = 16
NCHUNK = ROWS_W // CH
NEG_CH = CH * NEG
NEG_SLAB = 64
NSLAB = NEG_CH // NEG_SLAB
PDIM = 2 * DIM


def _log_sigmoid(x):
    e = jnp.exp(-jnp.abs(x))
    t = e / (2.0 + e)
    t2 = t * t
    ln = 2.0 * t * (1.0 + t2 * (1.0 / 3.0 + t2 * (0.2 + t2 * (1.0 / 7.0 + t2 * (1.0 / 9.0)))))
    return jnp.minimum(x, 0.0) - ln


def _sc_partial_loss(v_i, v_j, negflat, nodes2, ctx2):
    mesh = plsc.VectorSubcoreMesh(core_axis_name="c", subcore_axis_name="s")
    cp = pltpu.CompilerParams(needs_layout_passes=False)

    @functools.partial(
        pl.kernel,
        mesh=mesh,
        compiler_params=cp,
        out_type=jax.ShapeDtypeStruct((NWORK * LANES,), jnp.float32),
        scratch_types=[
            pltpu.VMEM((ROWS_W,), jnp.int32),
            pltpu.VMEM((ROWS_W,), jnp.int32),
            pltpu.VMEM((NEG_W,), jnp.int32),
            pltpu.VMEM((ROWS_W,), jnp.int32),
            pltpu.VMEM((ROWS_W,), jnp.int32),
            pltpu.VMEM((NEG_W,), jnp.int32),
            pltpu.VMEM((2, CH, PDIM), jnp.float32),
            pltpu.VMEM((2, CH, PDIM), jnp.float32),
            pltpu.VMEM((2, NEG_CH, PDIM), jnp.float32),
            pltpu.VMEM((LANES,), jnp.float32),
            pltpu.SemaphoreType.DMA,
            pltpu.SemaphoreType.DMA,
        ],
    )
    def sc_kernel(vi_hbm, vj_hbm, neg_hbm, nodes_hbm, ctx_hbm, out_hbm,
                  vi_idx, vj_idx, neg_idx, vi_pair, vj_pair, neg_pair,
                  vi_rows, vj_rows, neg_rows, loss_v, sem0, sem1):
        wid = lax.axis_index("s") * NCORE + lax.axis_index("c")
        lane = lax.iota(jnp.int32, LANES)

        pltpu.sync_copy(vi_hbm.at[pl.ds(wid * ROWS_W, ROWS_W)], vi_idx)
        pltpu.sync_copy(vj_hbm.at[pl.ds(wid * ROWS_W, ROWS_W)], vj_idx)
        pltpu.sync_copy(neg_hbm.at[pl.ds(wid * NEG_W, NEG_W)], neg_idx)

        @pl.loop(0, ROWS_W // LANES)
        def _(t):
            sl = pl.ds(t * LANES, LANES)
            vi_pair[sl] = vi_idx[sl] >> 1
            vj_pair[sl] = vj_idx[sl] >> 1

        @pl.loop(0, NEG_W // LANES)
        def _(t):
            sl = pl.ds(t * LANES, LANES)
            neg_pair[sl] = neg_idx[sl] >> 1

        def issue(c, b, sem):
            sl = pl.ds(c * CH, CH)
            pltpu.async_copy(nodes_hbm.at[vi_pair.at[sl]], vi_rows.at[b], sem)
            pltpu.async_copy(ctx_hbm.at[vj_pair.at[sl]], vj_rows.at[b], sem)
            for j in range(NSLAB):
                nsl = pl.ds(c * NEG_CH + j * NEG_SLAB, NEG_SLAB)
                pltpu.async_copy(ctx_hbm.at[neg_pair.at[nsl]],
                                 neg_rows.at[b].at[pl.ds(j * NEG_SLAB, NEG_SLAB)],
                                 sem)

        def drain(b, sem):
            pltpu.make_async_copy(nodes_hbm.at[pl.ds(0, CH)],
                                  vi_rows.at[b], sem).wait()
            pltpu.make_async_copy(nodes_hbm.at[pl.ds(0, CH)],
                                  vj_rows.at[b], sem).wait()
            pltpu.make_async_copy(nodes_hbm.at[pl.ds(0, NEG_CH)],
                                  neg_rows.at[b], sem).wait()

        def compute(c, b, loss):
            grow = c * CH + lane
            nbase = c * NEG_CH + lane * NEG
            vi_off = (plsc.load_gather(vi_idx, [grow]) & 1) * DIM
            vj_off = (plsc.load_gather(vj_idx, [grow]) & 1) * DIM
            ng_off = [(plsc.load_gather(neg_idx, [nbase + k]) & 1) * DIM
                      for k in range(NEG)]
            nrows = [lane * NEG + k for k in range(NEG)]
            vi_b, vj_b, ng_b = vi_rows.at[b], vj_rows.at[b], neg_rows.at[b]

            def dot_body(d, accs):
                vi_d = plsc.load_gather(vi_b, [lane, vi_off + d])
                vj_d = plsc.load_gather(vj_b, [lane, vj_off + d])
                out = [accs[0] + vi_d * vj_d]
                for k in range(NEG):
                    nv = plsc.load_gather(ng_b, [nrows[k], ng_off[k] + d])
                    out.append(accs[k + 1] + vi_d * nv)
                return tuple(out)

            zero = jnp.zeros((LANES,), jnp.float32)
            accs = lax.fori_loop(0, DIM, dot_body, (zero,) * (NEG + 1))
            ls = _log_sigmoid(accs[0])
            for k in range(NEG):
                ls = ls + _log_sigmoid(-accs[k + 1])
            return loss + ls

        issue(0, 0, sem0)
        issue(1, 1, sem1)

        def body(m, loss):
            c0 = 2 * m
            drain(0, sem0)
            loss = compute(c0, 0, loss)
            issue(jnp.minimum(c0 + 2, NCHUNK - 2), 0, sem0)
            drain(1, sem1)
            loss = compute(c0 + 1, 1, loss)
            issue(jnp.minimum(c0 + 3, NCHUNK - 1), 1, sem1)
            return loss

        loss = lax.fori_loop(0, NCHUNK // 2, body,
                             jnp.zeros((LANES,), jnp.float32))
        drain(0, sem0)
        drain(1, sem1)
        loss_v[...] = loss
        pltpu.sync_copy(loss_v, out_hbm.at[pl.ds(wid * LANES, LANES)])

    return sc_kernel(v_i, v_j, negflat, nodes2, ctx2)


def _tc_finish(partials):
    def body(p_ref, o_ref):
        o_ref[0, 0] = -jnp.sum(p_ref[...]) * (1.0 / BATCH)

    return pl.pallas_call(
        body,
        out_shape=jax.ShapeDtypeStruct((1, 1), jnp.float32),
        out_specs=pl.BlockSpec(memory_space=pltpu.SMEM),
    )(partials)


def kernel(v_i, v_j, negsamples, nodes_embeddings, context_nodes_embed):
    negflat = negsamples.reshape(BATCH * NEG)
    nodes2 = nodes_embeddings.reshape(SIZE // 2, PDIM)
    ctx2 = context_nodes_embed.reshape(SIZE // 2, PDIM)
    partials = _sc_partial_loss(v_i.astype(jnp.int32), v_j.astype(jnp.int32),
                                negflat.astype(jnp.int32), nodes2, ctx2)
    return _tc_finish(partials.reshape(NWORK, LANES))[0, 0]
